# baseline (speedup 1.0000x reference)
import jax
import jax.numpy as jnp
from jax.experimental import pallas as pl

_BM = 128


def _mv_block(a_ref, x_ref, o_ref):
    i = pl.program_id(0)
    o_ref[pl.ds(i * _BM, _BM), :] = jax.lax.dot_general(
        a_ref[...], x_ref[...],
        dimension_numbers=(((1,), (1,)), ((), ())),
        preferred_element_type=jnp.float32,
    )


def kernel(A, x):
    m, k = A.shape
    x2 = x.reshape(1, k)
    out = pl.pallas_call(
        _mv_block,
        grid=(m // _BM,),
        in_specs=[
            pl.BlockSpec((_BM, k), lambda i: (i, 0)),
            pl.BlockSpec((1, k), lambda i: (0, 0)),
        ],
        out_specs=pl.BlockSpec((m, 1), lambda i: (0, 0)),
        out_shape=jax.ShapeDtypeStruct((m, 1), jnp.float32),
    )(A, x2)
    return out.reshape(m)


# speedup vs baseline: 1.0206x; 1.0206x over previous
import jax
import jax.numpy as jnp
from jax.experimental import pallas as pl
from jax.experimental.pallas import tpu as pltpu

_BM = 256


def _mv_block(a_ref, x_ref, o_ref):
    i = pl.program_id(0)
    r = jax.lax.dot_general(
        a_ref[...], x_ref[...],
        dimension_numbers=(((1,), (1,)), ((), ())),
        preferred_element_type=jnp.float32,
    )
    o_ref[pl.ds(i, 1), :] = r.reshape(1, _BM)


def kernel(A, x):
    m, k = A.shape
    x2 = x.reshape(1, k)
    out = pl.pallas_call(
        _mv_block,
        grid=(m // _BM,),
        in_specs=[
            pl.BlockSpec((_BM, k), lambda i: (i, 0)),
            pl.BlockSpec((1, k), lambda i: (0, 0)),
        ],
        out_specs=pl.BlockSpec((m // _BM, _BM), lambda i: (0, 0)),
        out_shape=jax.ShapeDtypeStruct((m // _BM, _BM), jnp.float32),
    )(A, x2)
    return out.reshape(m)


# speedup vs baseline: 1.0248x; 1.0041x over previous
import jax
import jax.numpy as jnp
from jax.experimental import pallas as pl
from jax.experimental.pallas import tpu as pltpu

_BM = 128


def _mv_block(a_ref, x_ref, o_ref):
    i = pl.program_id(0)
    r = jax.lax.dot_general(
        a_ref[...], x_ref[...],
        dimension_numbers=(((1,), (1,)), ((), ())),
        preferred_element_type=jnp.float32,
    )
    o_ref[pl.ds(i, 1), :] = r.reshape(1, _BM)


def kernel(A, x):
    m, k = A.shape
    x2 = x.reshape(1, k)
    out = pl.pallas_call(
        _mv_block,
        grid=(m // _BM,),
        in_specs=[
            pl.BlockSpec((_BM, k), lambda i: (i, 0)),
            pl.BlockSpec((1, k), lambda i: (0, 0)),
        ],
        out_specs=pl.BlockSpec((m // _BM, _BM), lambda i: (0, 0)),
        out_shape=jax.ShapeDtypeStruct((m // _BM, _BM), jnp.float32),
    )(A, x2)
    return out.reshape(m)
